# baseline (speedup 1.0000x reference)
import functools

import jax
import jax.numpy as jnp
from jax import lax
from jax.experimental import pallas as pl
from jax.experimental.pallas import tpu as pltpu
from jax.experimental.pallas import tpu_sc as plsc

_NC = 2
_NS = 16
_NW = _NC * _NS
_BC = 2
_NBUF = 3
_B_SC = 192
_BB = 32


@functools.lru_cache(maxsize=None)
def _build_sc(B, T, V, dtype_name):
    dtype = jnp.dtype(dtype_name)
    BPW = B // _NW
    NCH = BPW // _BC

    mesh = plsc.VectorSubcoreMesh(core_axis_name="c", subcore_axis_name="s")

    @functools.partial(
        pl.kernel,
        mesh=mesh,
        compiler_params=pltpu.CompilerParams(use_tc_tiling_on_sc=False),
        out_type=jax.ShapeDtypeStruct((B, T, V), dtype),
        scratch_types=[
            pltpu.VMEM((BPW, T), jnp.int32),
            [pltpu.VMEM((_BC, T, V), dtype) for _ in range(_NBUF)],
            [pltpu.SemaphoreType.DMA for _ in range(_NBUF)],
            [pltpu.SemaphoreType.DMA for _ in range(_NBUF)],
        ],
    )
    def sc_gather(table_hbm, src_hbm, out_hbm, idx_v, bufs, gsems, ssems):
        wid = lax.axis_index("s") * _NC + lax.axis_index("c")
        base_b = wid * BPW
        pltpu.sync_copy(src_hbm.at[pl.ds(base_b, BPW)], idx_v)

        def gathers(c):
            k = c % _NBUF
            return [
                pltpu.make_async_copy(
                    table_hbm.at[idx_v.at[c * _BC + j]],
                    bufs[k].at[j], gsems[k])
                for j in range(_BC)
            ]

        def scatter(c):
            k = c % _NBUF
            return pltpu.make_async_copy(
                bufs[k],
                out_hbm.at[pl.ds(base_b + c * _BC, _BC)],
                ssems[k])

        for c in range(min(_NBUF, NCH)):
            for g in gathers(c):
                g.start()
        for c in range(NCH):
            for g in gathers(c):
                g.wait()
            scatter(c).start()
            if c + _NBUF < NCH:
                scatter(c).wait()
                for g in gathers(c + _NBUF):
                    g.start()
        for c in range(max(NCH - _NBUF, 0), NCH):
            scatter(c).wait()

    return sc_gather


@functools.lru_cache(maxsize=None)
def _build_tc_gather(B, T, V, Vr, b0, dtype_name):
    dtype = jnp.dtype(dtype_name)
    nb = B - b0

    def body(idx_ref, table_ref, out_ref):
        i = pl.program_id(0)
        zero = jnp.zeros((V,), dtype)
        for j in range(_BB):
            b = b0 + i * _BB + j
            out_ref[j, 0, :] = zero
            for t in range(1, T):
                row = idx_ref[b, t - 1]
                out_ref[j, t, :] = table_ref[row, :]

    return pl.pallas_call(
        body,
        grid_spec=pltpu.PrefetchScalarGridSpec(
            num_scalar_prefetch=1,
            grid=(nb // _BB,),
            in_specs=[pl.BlockSpec((Vr, V), lambda i, *_: (0, 0))],
            out_specs=pl.BlockSpec((_BB, T, V),
                                   lambda i, *_: (i + b0 // _BB, 0, 0)),
        ),
        out_shape=jax.ShapeDtypeStruct((B, T, V), dtype),
    )


@functools.lru_cache(maxsize=None)
def _build_tc_merge(B, T, V, nb, dtype_name):
    dtype = jnp.dtype(dtype_name)

    def body(mid_ref, full_ref, out_ref):
        del full_ref
        out_ref[...] = mid_ref[...]

    return pl.pallas_call(
        body,
        grid=(nb // _BB,),
        in_specs=[
            pl.BlockSpec((_BB, T, V), lambda i: (i, 0, 0)),
            pl.BlockSpec(memory_space=pl.ANY),
        ],
        out_specs=pl.BlockSpec((_BB, T, V), lambda i: (i, 0, 0)),
        out_shape=jax.ShapeDtypeStruct((B, T, V), dtype),
        input_output_aliases={1: 0},
    )


def kernel(idx, log_probs):
    B, T = idx.shape
    Vr, V = log_probs.shape
    idx = idx.astype(jnp.int32)
    dn = log_probs.dtype.name
    table_aug = jnp.concatenate(
        [log_probs, jnp.zeros((1, V), log_probs.dtype)], axis=0)
    src_sc = jnp.concatenate(
        [jnp.full((_B_SC, 1), Vr, jnp.int32), idx[:_B_SC, :-1]], axis=1)
    mid = _build_sc(_B_SC, T, V, dn)(table_aug, src_sc)
    out0 = _build_tc_gather(B, T, V, Vr, _B_SC, dn)(idx, log_probs)
    return _build_tc_merge(B, T, V, _B_SC, dn)(mid, out0)


# speedup vs baseline: 1.1163x; 1.1163x over previous
import functools

import jax
import jax.numpy as jnp
from jax import lax
from jax.experimental import pallas as pl
from jax.experimental.pallas import tpu as pltpu
from jax.experimental.pallas import tpu_sc as plsc

_NC = 2
_NS = 16
_NW = _NC * _NS
_BC = 2
_NBUF = 3
_B_SC = 64
_BB = 32


@functools.lru_cache(maxsize=None)
def _build_sc(B, T, V, dtype_name):
    dtype = jnp.dtype(dtype_name)
    BPW = B // _NW
    NCH = BPW // _BC

    mesh = plsc.VectorSubcoreMesh(core_axis_name="c", subcore_axis_name="s")

    @functools.partial(
        pl.kernel,
        mesh=mesh,
        compiler_params=pltpu.CompilerParams(use_tc_tiling_on_sc=False),
        out_type=jax.ShapeDtypeStruct((B, T, V), dtype),
        scratch_types=[
            pltpu.VMEM((BPW, T), jnp.int32),
            [pltpu.VMEM((_BC, T, V), dtype) for _ in range(_NBUF)],
            [pltpu.SemaphoreType.DMA for _ in range(_NBUF)],
            [pltpu.SemaphoreType.DMA for _ in range(_NBUF)],
        ],
    )
    def sc_gather(table_hbm, src_hbm, out_hbm, idx_v, bufs, gsems, ssems):
        wid = lax.axis_index("s") * _NC + lax.axis_index("c")
        base_b = wid * BPW
        pltpu.sync_copy(src_hbm.at[pl.ds(base_b, BPW)], idx_v)

        def gathers(c):
            k = c % _NBUF
            return [
                pltpu.make_async_copy(
                    table_hbm.at[idx_v.at[c * _BC + j]],
                    bufs[k].at[j], gsems[k])
                for j in range(_BC)
            ]

        def scatter(c):
            k = c % _NBUF
            return pltpu.make_async_copy(
                bufs[k],
                out_hbm.at[pl.ds(base_b + c * _BC, _BC)],
                ssems[k])

        for c in range(min(_NBUF, NCH)):
            for g in gathers(c):
                g.start()
        for c in range(NCH):
            for g in gathers(c):
                g.wait()
            scatter(c).start()
            if c + _NBUF < NCH:
                scatter(c).wait()
                for g in gathers(c + _NBUF):
                    g.start()
        for c in range(max(NCH - _NBUF, 0), NCH):
            scatter(c).wait()

    return sc_gather


@functools.lru_cache(maxsize=None)
def _build_tc_gather(B, T, V, Vr, b0, dtype_name):
    dtype = jnp.dtype(dtype_name)
    nb = B - b0

    def body(idx_ref, table_ref, out_ref):
        i = pl.program_id(0)
        zero = jnp.zeros((V,), dtype)
        for j in range(_BB):
            b = b0 + i * _BB + j
            out_ref[j, 0, :] = zero
            for t in range(1, T):
                row = idx_ref[b, t - 1]
                out_ref[j, t, :] = table_ref[row, :]

    return pl.pallas_call(
        body,
        grid_spec=pltpu.PrefetchScalarGridSpec(
            num_scalar_prefetch=1,
            grid=(nb // _BB,),
            in_specs=[pl.BlockSpec((Vr, V), lambda i, *_: (0, 0))],
            out_specs=pl.BlockSpec((_BB, T, V),
                                   lambda i, *_: (i + b0 // _BB, 0, 0)),
        ),
        out_shape=jax.ShapeDtypeStruct((B, T, V), dtype),
    )


@functools.lru_cache(maxsize=None)
def _build_tc_merge(B, T, V, nb, dtype_name):
    dtype = jnp.dtype(dtype_name)

    def body(mid_ref, full_ref, out_ref):
        del full_ref
        out_ref[...] = mid_ref[...]

    return pl.pallas_call(
        body,
        grid=(nb // _BB,),
        in_specs=[
            pl.BlockSpec((_BB, T, V), lambda i: (i, 0, 0)),
            pl.BlockSpec(memory_space=pl.ANY),
        ],
        out_specs=pl.BlockSpec((_BB, T, V), lambda i: (i, 0, 0)),
        out_shape=jax.ShapeDtypeStruct((B, T, V), dtype),
        input_output_aliases={1: 0},
    )


def kernel(idx, log_probs):
    B, T = idx.shape
    Vr, V = log_probs.shape
    idx = idx.astype(jnp.int32)
    dn = log_probs.dtype.name
    table_aug = jnp.concatenate(
        [log_probs, jnp.zeros((1, V), log_probs.dtype)], axis=0)
    src_sc = jnp.concatenate(
        [jnp.full((_B_SC, 1), Vr, jnp.int32), idx[:_B_SC, :-1]], axis=1)
    mid = _build_sc(_B_SC, T, V, dn)(table_aug, src_sc)
    out0 = _build_tc_gather(B, T, V, Vr, _B_SC, dn)(idx, log_probs)
    return _build_tc_merge(B, T, V, _B_SC, dn)(mid, out0)
